# baseline (speedup 1.0000x reference)
import functools

import jax
import jax.numpy as jnp
from jax import lax
from jax.experimental import pallas as pl
from jax.experimental.pallas import tpu as pltpu

B, L, DM, H, BS = 2, 4096, 1024, 16, 64
DH = DM // H
NB = L // BS
KK = 8
LB = 512
NPOOL = LB // BS

_INTERPRET = False


def _phi(x):
    return jnp.where(x > 0, x + 1.0, jnp.exp(x))


def _qkv_kernel(x_ref, wq_ref, wk_ref, wv_ref,
                q_ref, k_ref, v_ref, qp_ref, kp_ref):
    x = x_ref[0]
    q = jnp.dot(x, wq_ref[:, :], preferred_element_type=jnp.float32)
    k = jnp.dot(x, wk_ref[:, :], preferred_element_type=jnp.float32)
    v = jnp.dot(x, wv_ref[:, :], preferred_element_type=jnp.float32)
    scale = 1.0 / BS
    qp = jnp.stack([jnp.sum(q[j * BS:(j + 1) * BS, :], axis=0) * scale
                    for j in range(NPOOL)], axis=0)
    kp = jnp.stack([jnp.sum(k[j * BS:(j + 1) * BS, :], axis=0) * scale
                    for j in range(NPOOL)], axis=0)
    for h in range(H):
        sl = slice(h * DH, (h + 1) * DH)
        q_ref[0, h] = q[:, sl]
        k_ref[0, h] = k[:, sl]
        v_ref[0, h] = v[:, sl]
        qp_ref[0, h] = qp[:, sl]
        kp_ref[0, h] = kp[:, sl]


def _qkv(hidden, Wq, Wk, Wv):
    grid = (B, L // LB)
    w_spec = pl.BlockSpec((DM, DM), lambda b, i: (0, 0))
    return pl.pallas_call(
        _qkv_kernel,
        grid=grid,
        in_specs=[
            pl.BlockSpec((1, LB, DM), lambda b, i: (b, i, 0)),
            w_spec, w_spec, w_spec,
        ],
        out_specs=[
            pl.BlockSpec((1, H, LB, DH), lambda b, i: (b, 0, i, 0)),
            pl.BlockSpec((1, H, LB, DH), lambda b, i: (b, 0, i, 0)),
            pl.BlockSpec((1, H, LB, DH), lambda b, i: (b, 0, i, 0)),
            pl.BlockSpec((1, H, NPOOL, DH), lambda b, i: (b, 0, i, 0)),
            pl.BlockSpec((1, H, NPOOL, DH), lambda b, i: (b, 0, i, 0)),
        ],
        out_shape=[
            jax.ShapeDtypeStruct((B, H, L, DH), jnp.float32),
            jax.ShapeDtypeStruct((B, H, L, DH), jnp.float32),
            jax.ShapeDtypeStruct((B, H, L, DH), jnp.float32),
            jax.ShapeDtypeStruct((B, H, NB, DH), jnp.float32),
            jax.ShapeDtypeStruct((B, H, NB, DH), jnp.float32),
        ],
        interpret=_INTERPRET,
    )(hidden, Wq, Wk, Wv)


def _scores_kernel(qp_ref, kp_ref, rq_ref, rk_ref, s_ref):
    qp = jnp.dot(qp_ref[0, 0], rq_ref[:, :], preferred_element_type=jnp.float32)
    kp = jnp.dot(kp_ref[0, 0], rk_ref[:, :], preferred_element_type=jnp.float32)
    s_ref[0, 0] = lax.dot_general(qp, kp, (((1,), (1,)), ((), ())),
                                  preferred_element_type=jnp.float32) * (1.0 / 8.0)


def _scores(q_pool, k_pool, Rq, Rk):
    grid = (B, H)
    r_spec = pl.BlockSpec((DH, DH), lambda b, h: (0, 0))
    return pl.pallas_call(
        _scores_kernel,
        grid=grid,
        in_specs=[
            pl.BlockSpec((1, 1, NB, DH), lambda b, h: (b, h, 0, 0)),
            pl.BlockSpec((1, 1, NB, DH), lambda b, h: (b, h, 0, 0)),
            r_spec, r_spec,
        ],
        out_specs=pl.BlockSpec((1, 1, NB, NB), lambda b, h: (b, h, 0, 0)),
        out_shape=jax.ShapeDtypeStruct((B, H, NB, NB), jnp.float32),
        interpret=_INTERPRET,
    )(q_pool, k_pool, Rq, Rk)


NROWS = B * H * NB
NWORK = 32
RW = NROWS // NWORK


def _topk_sc_kernel(s_hbm, sel_hbm, s_v, o_v, kbuf, vbuf):
    from jax.experimental.pallas import tpu_sc as plsc
    wid = lax.axis_index("s") * 2 + lax.axis_index("c")
    base = wid * RW
    pltpu.sync_copy(s_hbm.at[pl.ds(base, RW)], s_v)

    idx16 = lax.iota(jnp.int32, 16)

    def row(i, _):
        for pair, dst in ((0, 0), (1, 8)):
            for c, off in ((2 * pair, 16), (2 * pair + 1, 24)):
                ks, vs = plsc.sort_key_val(
                    s_v[i, pl.ds(c * 16, 16)], idx16 + 16 * c,
                    descending=True)
                kbuf[pl.ds(off, 16)] = ks
                vbuf[pl.ds(off, 16)] = vs
            km, vm = plsc.sort_key_val(
                kbuf[pl.ds(16, 16)], vbuf[pl.ds(16, 16)], descending=True)
            kbuf[pl.ds(dst, 16)] = km
            vbuf[pl.ds(dst, 16)] = vm
        _, vf = plsc.sort_key_val(
            kbuf[pl.ds(0, 16)], vbuf[pl.ds(0, 16)], descending=True)
        o_v[i, :] = vf
        return 0

    lax.fori_loop(0, RW, row, 0)
    pltpu.sync_copy(o_v, sel_hbm.at[pl.ds(base, RW)])


def _topk_sc(scores_flat):
    import functools as _ft
    from jax.experimental.pallas import tpu_sc as plsc
    mesh = plsc.VectorSubcoreMesh(core_axis_name="c", subcore_axis_name="s")
    kern = _ft.partial(
        pl.kernel,
        mesh=mesh,
        out_type=jax.ShapeDtypeStruct((NROWS, 16), jnp.int32),
        scratch_types=[
            pltpu.VMEM((RW, NB), jnp.float32),
            pltpu.VMEM((RW, 16), jnp.int32),
            pltpu.VMEM((40,), jnp.float32),
            pltpu.VMEM((40,), jnp.int32),
        ],
        compiler_params=pltpu.CompilerParams(needs_layout_passes=False),
    )(_topk_sc_kernel)
    return kern(scores_flat)


def _linstate_kernel(k_ref, v_ref, kv_ref, z_ref):
    pk = _phi(k_ref[0, 0])
    kv_ref[0, 0] = lax.dot_general(
        pk, v_ref[0, 0], (((0,), (0,)), ((), ())),
        preferred_element_type=jnp.float32)
    z_ref[0, 0] = jnp.sum(pk, axis=0, keepdims=True)


def _linstate(k, v):
    grid = (B, H)
    return pl.pallas_call(
        _linstate_kernel,
        grid=grid,
        in_specs=[
            pl.BlockSpec((1, 1, L, DH), lambda b, h: (b, h, 0, 0)),
            pl.BlockSpec((1, 1, L, DH), lambda b, h: (b, h, 0, 0)),
        ],
        out_specs=[
            pl.BlockSpec((1, 1, DH, DH), lambda b, h: (b, h, 0, 0)),
            pl.BlockSpec((1, 1, 1, DH), lambda b, h: (b, h, 0, 0)),
        ],
        out_shape=[
            jax.ShapeDtypeStruct((B, H, DH, DH), jnp.float32),
            jax.ShapeDtypeStruct((B, H, 1, DH), jnp.float32),
        ],
        interpret=_INTERPRET,
    )(k, v)


GQ = 8


def _attn_kernel(sel_ref, alpha_ref, q_ref, k_ref, v_ref, kv_ref, z_ref,
                 o_ref):
    b = pl.program_id(0)
    h = pl.program_id(1)
    qi = pl.program_id(2)

    a = alpha_ref[h]
    kv = kv_ref[0, 0]
    z = z_ref[0, 0]
    for g in range(GQ):
        q_blk = q_ref[0, 0, g * BS:(g + 1) * BS, :]
        base = ((b * H + h) * NB + qi * GQ + g) * KK
        s = jnp.concatenate(
            [lax.dot_general(q_blk, k_ref[0, 0, pl.ds(sel_ref[base + j] * BS, BS), :],
                             (((1,), (1,)), ((), ())),
                             preferred_element_type=jnp.float32)
             for j in range(KK)], axis=1)
        m = jnp.max(s, axis=1, keepdims=True)
        e = jnp.exp(s - m)
        den_s = jnp.sum(e, axis=1, keepdims=True)
        o_s = sum(
            jnp.dot(e[:, j * BS:(j + 1) * BS],
                    v_ref[0, 0, pl.ds(sel_ref[base + j] * BS, BS), :],
                    preferred_element_type=jnp.float32)
            for j in range(KK)) / den_s

        pq = _phi(q_blk)
        num = jnp.dot(pq, kv, preferred_element_type=jnp.float32)
        den_l = jnp.sum(pq * z, axis=1, keepdims=True) + 1e-6
        o_l = num / den_l

        o_ref[0, 0, g * BS:(g + 1) * BS, :] = a * o_s + (1.0 - a) * o_l


def _attention(q, k, v, kv, z, sel_flat, alpha):
    grid = (B, H, NB // GQ)
    grid_spec = pltpu.PrefetchScalarGridSpec(
        num_scalar_prefetch=2,
        grid=grid,
        in_specs=[
            pl.BlockSpec((1, 1, GQ * BS, DH), lambda b, h, qi, *_: (b, h, qi, 0)),
            pl.BlockSpec((1, 1, L, DH), lambda b, h, qi, *_: (b, h, 0, 0)),
            pl.BlockSpec((1, 1, L, DH), lambda b, h, qi, *_: (b, h, 0, 0)),
            pl.BlockSpec((1, 1, DH, DH), lambda b, h, qi, *_: (b, h, 0, 0)),
            pl.BlockSpec((1, 1, 1, DH), lambda b, h, qi, *_: (b, h, 0, 0)),
        ],
        out_specs=pl.BlockSpec((1, 1, GQ * BS, DH),
                               lambda b, h, qi, *_: (b, h, qi, 0)),
    )
    return pl.pallas_call(
        _attn_kernel,
        grid_spec=grid_spec,
        out_shape=jax.ShapeDtypeStruct((B, H, L, DH), jnp.float32),
        interpret=_INTERPRET,
    )(sel_flat, alpha, q, k, v, kv, z)


def _proj_kernel(x_ref, w_ref, o_ref):
    x = jnp.concatenate([x_ref[0, h] for h in range(H)], axis=1)
    o_ref[0] = jnp.dot(x, w_ref[:, :], preferred_element_type=jnp.float32)


def _out_proj(x, Wo):
    grid = (B, L // LB)
    return pl.pallas_call(
        _proj_kernel,
        grid=grid,
        in_specs=[
            pl.BlockSpec((1, H, LB, DH), lambda b, i: (b, 0, i, 0)),
            pl.BlockSpec((DM, DM), lambda b, i: (0, 0)),
        ],
        out_specs=pl.BlockSpec((1, LB, DM), lambda b, i: (b, i, 0)),
        out_shape=jax.ShapeDtypeStruct((B, L, DM), jnp.float32),
        interpret=_INTERPRET,
    )(x, Wo)


@jax.jit
def kernel(hidden_states, Wq, Wk, Wv, Wo, Rq, Rk, alpha_logits):
    q, k, v, q_pool, k_pool = _qkv(hidden_states, Wq, Wk, Wv)
    scores = _scores(q_pool, k_pool, Rq, Rk)
    sel_pad = _topk_sc(scores.reshape(NROWS, NB))
    sel_flat = sel_pad[:, :KK].reshape(-1)
    alpha = jax.nn.sigmoid(alpha_logits).reshape(H)
    kv, z = _linstate(k, v)
    out_attn = _attention(q, k, v, kv, z, sel_flat, alpha)
    return _out_proj(out_attn, Wo)


# speedup vs baseline: 1.3395x; 1.3395x over previous
import functools

import jax
import jax.numpy as jnp
from jax import lax
from jax.experimental import pallas as pl
from jax.experimental.pallas import tpu as pltpu

B, L, DM, H, BS = 2, 4096, 1024, 16, 64
DH = DM // H
NB = L // BS
KK = 8
LB = 512
NPOOL = LB // BS

_INTERPRET = False


def _phi(x):
    return jnp.where(x > 0, x + 1.0, jnp.exp(x))


def _qkv_kernel(x_ref, wq_ref, wk_ref, wv_ref,
                q_ref, k_ref, v_ref, qp_ref, kp_ref):
    x = x_ref[0]
    q = jnp.dot(x, wq_ref[:, :], preferred_element_type=jnp.float32)
    k = jnp.dot(x, wk_ref[:, :], preferred_element_type=jnp.float32)
    v = jnp.dot(x, wv_ref[:, :], preferred_element_type=jnp.float32)
    scale = 1.0 / BS
    qp = jnp.stack([jnp.sum(q[j * BS:(j + 1) * BS, :], axis=0) * scale
                    for j in range(NPOOL)], axis=0)
    kp = jnp.stack([jnp.sum(k[j * BS:(j + 1) * BS, :], axis=0) * scale
                    for j in range(NPOOL)], axis=0)
    for h in range(H):
        sl = slice(h * DH, (h + 1) * DH)
        q_ref[0, h] = q[:, sl]
        k_ref[0, h] = k[:, sl]
        v_ref[0, h] = v[:, sl]
        qp_ref[0, h] = qp[:, sl]
        kp_ref[0, h] = kp[:, sl]


def _qkv(hidden, Wq, Wk, Wv):
    grid = (B, L // LB)
    w_spec = pl.BlockSpec((DM, DM), lambda b, i: (0, 0))
    return pl.pallas_call(
        _qkv_kernel,
        grid=grid,
        in_specs=[
            pl.BlockSpec((1, LB, DM), lambda b, i: (b, i, 0)),
            w_spec, w_spec, w_spec,
        ],
        out_specs=[
            pl.BlockSpec((1, H, LB, DH), lambda b, i: (b, 0, i, 0)),
            pl.BlockSpec((1, H, LB, DH), lambda b, i: (b, 0, i, 0)),
            pl.BlockSpec((1, H, LB, DH), lambda b, i: (b, 0, i, 0)),
            pl.BlockSpec((1, H, NPOOL, DH), lambda b, i: (b, 0, i, 0)),
            pl.BlockSpec((1, H, NPOOL, DH), lambda b, i: (b, 0, i, 0)),
        ],
        out_shape=[
            jax.ShapeDtypeStruct((B, H, L, DH), jnp.float32),
            jax.ShapeDtypeStruct((B, H, L, DH), jnp.float32),
            jax.ShapeDtypeStruct((B, H, L, DH), jnp.float32),
            jax.ShapeDtypeStruct((B, H, NB, DH), jnp.float32),
            jax.ShapeDtypeStruct((B, H, NB, DH), jnp.float32),
        ],
        interpret=_INTERPRET,
    )(hidden, Wq, Wk, Wv)


def _scores_kernel(qp_ref, kp_ref, rq_ref, rk_ref, s_ref):
    qp = jnp.dot(qp_ref[0, 0], rq_ref[:, :], preferred_element_type=jnp.float32)
    kp = jnp.dot(kp_ref[0, 0], rk_ref[:, :], preferred_element_type=jnp.float32)
    s_ref[0, 0] = lax.dot_general(qp, kp, (((1,), (1,)), ((), ())),
                                  preferred_element_type=jnp.float32) * (1.0 / 8.0)


def _scores(q_pool, k_pool, Rq, Rk):
    grid = (B, H)
    r_spec = pl.BlockSpec((DH, DH), lambda b, h: (0, 0))
    return pl.pallas_call(
        _scores_kernel,
        grid=grid,
        in_specs=[
            pl.BlockSpec((1, 1, NB, DH), lambda b, h: (b, h, 0, 0)),
            pl.BlockSpec((1, 1, NB, DH), lambda b, h: (b, h, 0, 0)),
            r_spec, r_spec,
        ],
        out_specs=pl.BlockSpec((1, 1, NB, NB), lambda b, h: (b, h, 0, 0)),
        out_shape=jax.ShapeDtypeStruct((B, H, NB, NB), jnp.float32),
        interpret=_INTERPRET,
    )(q_pool, k_pool, Rq, Rk)


NROWS = B * H * NB
NWORK = 32
RW = NROWS // NWORK


def _topk_sc_kernel(s_hbm, sel_hbm, s_v, o_v, kbuf, vbuf):
    from jax.experimental.pallas import tpu_sc as plsc
    wid = lax.axis_index("s") * 2 + lax.axis_index("c")
    base = wid * RW
    pltpu.sync_copy(s_hbm.at[pl.ds(base, RW)], s_v)

    idx16 = lax.iota(jnp.int32, 16)

    def row(i, _):
        for pair, dst in ((0, 0), (1, 8)):
            for c, off in ((2 * pair, 16), (2 * pair + 1, 24)):
                ks, vs = plsc.sort_key_val(
                    s_v[i, pl.ds(c * 16, 16)], idx16 + 16 * c,
                    descending=True)
                kbuf[pl.ds(off, 16)] = ks
                vbuf[pl.ds(off, 16)] = vs
            km, vm = plsc.sort_key_val(
                kbuf[pl.ds(16, 16)], vbuf[pl.ds(16, 16)], descending=True)
            kbuf[pl.ds(dst, 16)] = km
            vbuf[pl.ds(dst, 16)] = vm
        _, vf = plsc.sort_key_val(
            kbuf[pl.ds(0, 16)], vbuf[pl.ds(0, 16)], descending=True)
        o_v[i, :] = vf
        return 0

    lax.fori_loop(0, RW, row, 0)
    pltpu.sync_copy(o_v, sel_hbm.at[pl.ds(base, RW)])


def _topk_sc(scores_flat):
    import functools as _ft
    from jax.experimental.pallas import tpu_sc as plsc
    mesh = plsc.VectorSubcoreMesh(core_axis_name="c", subcore_axis_name="s")
    kern = _ft.partial(
        pl.kernel,
        mesh=mesh,
        out_type=jax.ShapeDtypeStruct((NROWS, 16), jnp.int32),
        scratch_types=[
            pltpu.VMEM((RW, NB), jnp.float32),
            pltpu.VMEM((RW, 16), jnp.int32),
            pltpu.VMEM((40,), jnp.float32),
            pltpu.VMEM((40,), jnp.int32),
        ],
        compiler_params=pltpu.CompilerParams(needs_layout_passes=False),
    )(_topk_sc_kernel)
    return kern(scores_flat)


def _linstate_kernel(k_ref, v_ref, kv_ref, z_ref):
    pk = _phi(k_ref[0, 0])
    kv_ref[0, 0] = lax.dot_general(
        pk, v_ref[0, 0], (((0,), (0,)), ((), ())),
        preferred_element_type=jnp.float32)
    z_ref[0, 0] = jnp.sum(pk, axis=0, keepdims=True)


def _linstate(k, v):
    grid = (B, H)
    return pl.pallas_call(
        _linstate_kernel,
        grid=grid,
        in_specs=[
            pl.BlockSpec((1, 1, L, DH), lambda b, h: (b, h, 0, 0)),
            pl.BlockSpec((1, 1, L, DH), lambda b, h: (b, h, 0, 0)),
        ],
        out_specs=[
            pl.BlockSpec((1, 1, DH, DH), lambda b, h: (b, h, 0, 0)),
            pl.BlockSpec((1, 1, 1, DH), lambda b, h: (b, h, 0, 0)),
        ],
        out_shape=[
            jax.ShapeDtypeStruct((B, H, DH, DH), jnp.float32),
            jax.ShapeDtypeStruct((B, H, 1, DH), jnp.float32),
        ],
        interpret=_INTERPRET,
    )(k, v)


GQ = 4


def _attn_kernel(sel_ref, alpha_ref, q_ref, k_ref, v_ref, kv_ref, z_ref,
                 o_ref):
    b = pl.program_id(0)
    h = pl.program_id(1)
    qi = pl.program_id(2)

    a = alpha_ref[h]
    kv = kv_ref[0, 0]
    z = z_ref[0, 0]
    for g in range(GQ):
        q_blk = q_ref[0, 0, g * BS:(g + 1) * BS, :]
        base = ((b * H + h) * NB + qi * GQ + g) * KK
        k_rows = []
        v_rows = []
        for j in range(KK):
            idx = sel_ref[base + j]
            k_rows.append(k_ref[0, 0, pl.ds(idx * BS, BS), :])
            v_rows.append(v_ref[0, 0, pl.ds(idx * BS, BS), :])
        k_cat = jnp.concatenate(k_rows, axis=0)
        v_cat = jnp.concatenate(v_rows, axis=0)
        s = lax.dot_general(q_blk, k_cat, (((1,), (1,)), ((), ())),
                            preferred_element_type=jnp.float32)
        m = jnp.max(s, axis=1, keepdims=True)
        e = jnp.exp(s - m)
        den_s = jnp.sum(e, axis=1, keepdims=True)
        o_s = jnp.dot(e, v_cat, preferred_element_type=jnp.float32) / den_s

        pq = _phi(q_blk)
        num = jnp.dot(pq, kv, preferred_element_type=jnp.float32)
        den_l = jnp.sum(pq * z, axis=1, keepdims=True) + 1e-6
        o_l = num / den_l

        o_ref[0, 0, g * BS:(g + 1) * BS, :] = a * o_s + (1.0 - a) * o_l


def _attention(q, k, v, kv, z, sel_flat, alpha):
    grid = (B, H, NB // GQ)
    grid_spec = pltpu.PrefetchScalarGridSpec(
        num_scalar_prefetch=2,
        grid=grid,
        in_specs=[
            pl.BlockSpec((1, 1, GQ * BS, DH), lambda b, h, qi, *_: (b, h, qi, 0)),
            pl.BlockSpec((1, 1, L, DH), lambda b, h, qi, *_: (b, h, 0, 0)),
            pl.BlockSpec((1, 1, L, DH), lambda b, h, qi, *_: (b, h, 0, 0)),
            pl.BlockSpec((1, 1, DH, DH), lambda b, h, qi, *_: (b, h, 0, 0)),
            pl.BlockSpec((1, 1, 1, DH), lambda b, h, qi, *_: (b, h, 0, 0)),
        ],
        out_specs=pl.BlockSpec((1, 1, GQ * BS, DH),
                               lambda b, h, qi, *_: (b, h, qi, 0)),
    )
    return pl.pallas_call(
        _attn_kernel,
        grid_spec=grid_spec,
        out_shape=jax.ShapeDtypeStruct((B, H, L, DH), jnp.float32),
        interpret=_INTERPRET,
    )(sel_flat, alpha, q, k, v, kv, z)


def _proj_kernel(x_ref, w_ref, o_ref):
    o_ref[0] = sum(
        jnp.dot(x_ref[0, h], w_ref[h], preferred_element_type=jnp.float32)
        for h in range(H))


def _out_proj(x, Wo):
    grid = (B, L // LB)
    return pl.pallas_call(
        _proj_kernel,
        grid=grid,
        in_specs=[
            pl.BlockSpec((1, H, LB, DH), lambda b, i: (b, 0, i, 0)),
            pl.BlockSpec((H, DH, DM), lambda b, i: (0, 0, 0)),
        ],
        out_specs=pl.BlockSpec((1, LB, DM), lambda b, i: (b, i, 0)),
        out_shape=jax.ShapeDtypeStruct((B, L, DM), jnp.float32),
        interpret=_INTERPRET,
    )(x, Wo.reshape(H, DH, DM))


@jax.jit
def kernel(hidden_states, Wq, Wk, Wv, Wo, Rq, Rk, alpha_logits):
    q, k, v, q_pool, k_pool = _qkv(hidden_states, Wq, Wk, Wv)
    scores = _scores(q_pool, k_pool, Rq, Rk)
    sel_pad = _topk_sc(scores.reshape(NROWS, NB))
    sel_flat = sel_pad[:, :KK].reshape(-1)
    alpha = jax.nn.sigmoid(alpha_logits).reshape(H)
    kv, z = _linstate(k, v)
    out_attn = _attention(q, k, v, kv, z, sel_flat, alpha)
    return _out_proj(out_attn, Wo)


# speedup vs baseline: 1.4469x; 1.0802x over previous
import functools

import jax
import jax.numpy as jnp
from jax import lax
from jax.experimental import pallas as pl
from jax.experimental.pallas import tpu as pltpu

B, L, DM, H, BS = 2, 4096, 1024, 16, 64
DH = DM // H
NB = L // BS
KK = 8
LB = 512
NPOOL = LB // BS

_INTERPRET = False


def _phi(x):
    return jnp.where(x > 0, x + 1.0, jnp.exp(x))


def _qkv_kernel(x_ref, wq_ref, wk_ref, wv_ref,
                q_ref, k_ref, v_ref, qp_ref, kp_ref):
    x = x_ref[0]
    q = jnp.dot(x, wq_ref[:, :], preferred_element_type=jnp.float32)
    k = jnp.dot(x, wk_ref[:, :], preferred_element_type=jnp.float32)
    v = jnp.dot(x, wv_ref[:, :], preferred_element_type=jnp.float32)
    scale = 1.0 / BS
    qp = jnp.stack([jnp.sum(q[j * BS:(j + 1) * BS, :], axis=0) * scale
                    for j in range(NPOOL)], axis=0)
    kp = jnp.stack([jnp.sum(k[j * BS:(j + 1) * BS, :], axis=0) * scale
                    for j in range(NPOOL)], axis=0)
    for h in range(H):
        sl = slice(h * DH, (h + 1) * DH)
        q_ref[0, h] = q[:, sl].astype(jnp.bfloat16)
        k_ref[0, h] = k[:, sl].astype(jnp.bfloat16)
        v_ref[0, h] = v[:, sl].astype(jnp.bfloat16)
        qp_ref[0, h] = qp[:, sl]
        kp_ref[0, h] = kp[:, sl]


def _qkv(hidden, Wq, Wk, Wv):
    grid = (B, L // LB)
    w_spec = pl.BlockSpec((DM, DM), lambda b, i: (0, 0))
    return pl.pallas_call(
        _qkv_kernel,
        grid=grid,
        in_specs=[
            pl.BlockSpec((1, LB, DM), lambda b, i: (b, i, 0)),
            w_spec, w_spec, w_spec,
        ],
        out_specs=[
            pl.BlockSpec((1, H, LB, DH), lambda b, i: (b, 0, i, 0)),
            pl.BlockSpec((1, H, LB, DH), lambda b, i: (b, 0, i, 0)),
            pl.BlockSpec((1, H, LB, DH), lambda b, i: (b, 0, i, 0)),
            pl.BlockSpec((1, H, NPOOL, DH), lambda b, i: (b, 0, i, 0)),
            pl.BlockSpec((1, H, NPOOL, DH), lambda b, i: (b, 0, i, 0)),
        ],
        out_shape=[
            jax.ShapeDtypeStruct((B, H, L, DH), jnp.bfloat16),
            jax.ShapeDtypeStruct((B, H, L, DH), jnp.bfloat16),
            jax.ShapeDtypeStruct((B, H, L, DH), jnp.bfloat16),
            jax.ShapeDtypeStruct((B, H, NB, DH), jnp.float32),
            jax.ShapeDtypeStruct((B, H, NB, DH), jnp.float32),
        ],
        interpret=_INTERPRET,
    )(hidden, Wq, Wk, Wv)


def _scores_kernel(qp_ref, kp_ref, rq_ref, rk_ref, s_ref):
    qp = jnp.dot(qp_ref[0, 0], rq_ref[:, :], preferred_element_type=jnp.float32)
    kp = jnp.dot(kp_ref[0, 0], rk_ref[:, :], preferred_element_type=jnp.float32)
    s_ref[0, 0] = lax.dot_general(qp, kp, (((1,), (1,)), ((), ())),
                                  preferred_element_type=jnp.float32) * (1.0 / 8.0)


def _scores(q_pool, k_pool, Rq, Rk):
    grid = (B, H)
    r_spec = pl.BlockSpec((DH, DH), lambda b, h: (0, 0))
    return pl.pallas_call(
        _scores_kernel,
        grid=grid,
        in_specs=[
            pl.BlockSpec((1, 1, NB, DH), lambda b, h: (b, h, 0, 0)),
            pl.BlockSpec((1, 1, NB, DH), lambda b, h: (b, h, 0, 0)),
            r_spec, r_spec,
        ],
        out_specs=pl.BlockSpec((1, 1, NB, NB), lambda b, h: (b, h, 0, 0)),
        out_shape=jax.ShapeDtypeStruct((B, H, NB, NB), jnp.float32),
        interpret=_INTERPRET,
    )(q_pool, k_pool, Rq, Rk)


NROWS = B * H * NB
NWORK = 32
RW = NROWS // NWORK


def _topk_sc_kernel(s_hbm, sel_hbm, s_v, o_v, kbuf, vbuf):
    from jax.experimental.pallas import tpu_sc as plsc
    wid = lax.axis_index("s") * 2 + lax.axis_index("c")
    base = wid * RW
    pltpu.sync_copy(s_hbm.at[pl.ds(base, RW)], s_v)

    idx16 = lax.iota(jnp.int32, 16)

    def row(i, _):
        for pair, dst in ((0, 0), (1, 8)):
            for c, off in ((2 * pair, 16), (2 * pair + 1, 24)):
                ks, vs = plsc.sort_key_val(
                    s_v[i, pl.ds(c * 16, 16)], idx16 + 16 * c,
                    descending=True)
                kbuf[pl.ds(off, 16)] = ks
                vbuf[pl.ds(off, 16)] = vs
            km, vm = plsc.sort_key_val(
                kbuf[pl.ds(16, 16)], vbuf[pl.ds(16, 16)], descending=True)
            kbuf[pl.ds(dst, 16)] = km
            vbuf[pl.ds(dst, 16)] = vm
        _, vf = plsc.sort_key_val(
            kbuf[pl.ds(0, 16)], vbuf[pl.ds(0, 16)], descending=True)
        o_v[i, :] = vf
        return 0

    lax.fori_loop(0, RW, row, 0)
    pltpu.sync_copy(o_v, sel_hbm.at[pl.ds(base, RW)])


def _topk_sc(scores_flat):
    import functools as _ft
    from jax.experimental.pallas import tpu_sc as plsc
    mesh = plsc.VectorSubcoreMesh(core_axis_name="c", subcore_axis_name="s")
    kern = _ft.partial(
        pl.kernel,
        mesh=mesh,
        out_type=jax.ShapeDtypeStruct((NROWS, 16), jnp.int32),
        scratch_types=[
            pltpu.VMEM((RW, NB), jnp.float32),
            pltpu.VMEM((RW, 16), jnp.int32),
            pltpu.VMEM((40,), jnp.float32),
            pltpu.VMEM((40,), jnp.int32),
        ],
        compiler_params=pltpu.CompilerParams(needs_layout_passes=False),
    )(_topk_sc_kernel)
    return kern(scores_flat)


def _linstate_kernel(k_ref, v_ref, kv_ref, z_ref):
    pk = _phi(k_ref[0, 0].astype(jnp.float32))
    kv_ref[0, 0] = lax.dot_general(
        pk.astype(jnp.bfloat16), v_ref[0, 0], (((0,), (0,)), ((), ())),
        preferred_element_type=jnp.float32)
    z_ref[0, 0] = jnp.sum(pk, axis=0, keepdims=True)


def _linstate(k, v):
    grid = (B, H)
    return pl.pallas_call(
        _linstate_kernel,
        grid=grid,
        in_specs=[
            pl.BlockSpec((1, 1, L, DH), lambda b, h: (b, h, 0, 0)),
            pl.BlockSpec((1, 1, L, DH), lambda b, h: (b, h, 0, 0)),
        ],
        out_specs=[
            pl.BlockSpec((1, 1, DH, DH), lambda b, h: (b, h, 0, 0)),
            pl.BlockSpec((1, 1, 1, DH), lambda b, h: (b, h, 0, 0)),
        ],
        out_shape=[
            jax.ShapeDtypeStruct((B, H, DH, DH), jnp.float32),
            jax.ShapeDtypeStruct((B, H, 1, DH), jnp.float32),
        ],
        interpret=_INTERPRET,
    )(k, v)


GQ = 4


def _attn_kernel(sel_ref, alpha_ref, q_ref, k_ref, v_ref, kv_ref, z_ref,
                 o_ref):
    b = pl.program_id(0)
    h = pl.program_id(1)
    qi = pl.program_id(2)

    a = alpha_ref[h]
    kv = kv_ref[0, 0]
    z = z_ref[0, 0]
    for g in range(GQ):
        q_blk = q_ref[0, 0, g * BS:(g + 1) * BS, :]
        base = ((b * H + h) * NB + qi * GQ + g) * KK
        k_rows = []
        v_rows = []
        for j in range(KK):
            idx = sel_ref[base + j]
            k_rows.append(k_ref[0, 0, pl.ds(idx * BS, BS), :])
            v_rows.append(v_ref[0, 0, pl.ds(idx * BS, BS), :])
        k_cat = jnp.concatenate(k_rows, axis=0)
        v_cat = jnp.concatenate(v_rows, axis=0)
        s = lax.dot_general(q_blk, k_cat, (((1,), (1,)), ((), ())),
                            preferred_element_type=jnp.float32)
        m = jnp.max(s, axis=1, keepdims=True)
        e = jnp.exp(s - m)
        den_s = jnp.sum(e, axis=1, keepdims=True)
        o_s = jnp.dot(e.astype(jnp.bfloat16), v_cat,
                      preferred_element_type=jnp.float32) / den_s

        pq = _phi(q_blk.astype(jnp.float32))
        num = jnp.dot(pq, kv, preferred_element_type=jnp.float32)
        den_l = jnp.sum(pq * z, axis=1, keepdims=True) + 1e-6
        o_l = num / den_l

        o_ref[0, 0, g * BS:(g + 1) * BS, :] = a * o_s + (1.0 - a) * o_l


def _attention(q, k, v, kv, z, sel_flat, alpha):
    grid = (B, H, NB // GQ)
    grid_spec = pltpu.PrefetchScalarGridSpec(
        num_scalar_prefetch=2,
        grid=grid,
        in_specs=[
            pl.BlockSpec((1, 1, GQ * BS, DH), lambda b, h, qi, *_: (b, h, qi, 0)),
            pl.BlockSpec((1, 1, L, DH), lambda b, h, qi, *_: (b, h, 0, 0)),
            pl.BlockSpec((1, 1, L, DH), lambda b, h, qi, *_: (b, h, 0, 0)),
            pl.BlockSpec((1, 1, DH, DH), lambda b, h, qi, *_: (b, h, 0, 0)),
            pl.BlockSpec((1, 1, 1, DH), lambda b, h, qi, *_: (b, h, 0, 0)),
        ],
        out_specs=pl.BlockSpec((1, 1, GQ * BS, DH),
                               lambda b, h, qi, *_: (b, h, qi, 0)),
    )
    return pl.pallas_call(
        _attn_kernel,
        grid_spec=grid_spec,
        out_shape=jax.ShapeDtypeStruct((B, H, L, DH), jnp.float32),
        interpret=_INTERPRET,
    )(sel_flat, alpha, q, k, v, kv, z)


def _proj_kernel(x_ref, w_ref, o_ref):
    x = jnp.concatenate([x_ref[0, h] for h in range(H)], axis=1)
    o_ref[0] = jnp.dot(x, w_ref[:, :], preferred_element_type=jnp.float32)


def _out_proj(x, Wo):
    grid = (B, L // LB)
    return pl.pallas_call(
        _proj_kernel,
        grid=grid,
        in_specs=[
            pl.BlockSpec((1, H, LB, DH), lambda b, i: (b, 0, i, 0)),
            pl.BlockSpec((DM, DM), lambda b, i: (0, 0)),
        ],
        out_specs=pl.BlockSpec((1, LB, DM), lambda b, i: (b, i, 0)),
        out_shape=jax.ShapeDtypeStruct((B, L, DM), jnp.float32),
        interpret=_INTERPRET,
    )(x, Wo)


@jax.jit
def kernel(hidden_states, Wq, Wk, Wv, Wo, Rq, Rk, alpha_logits):
    q, k, v, q_pool, k_pool = _qkv(hidden_states, Wq, Wk, Wv)
    scores = _scores(q_pool, k_pool, Rq, Rk)
    sel_pad = _topk_sc(scores.reshape(NROWS, NB))
    sel_flat = sel_pad[:, :KK].reshape(-1)
    alpha = jax.nn.sigmoid(alpha_logits).reshape(H)
    kv, z = _linstate(k, v)
    out_attn = _attention(q, k, v, kv, z, sel_flat, alpha)
    return _out_proj(out_attn, Wo)


# speedup vs baseline: 1.4561x; 1.0063x over previous
import functools

import jax
import jax.numpy as jnp
from jax import lax
from jax.experimental import pallas as pl
from jax.experimental.pallas import tpu as pltpu

B, L, DM, H, BS = 2, 4096, 1024, 16, 64
DH = DM // H
NB = L // BS
KK = 8
LB = 512
NPOOL = LB // BS

_INTERPRET = False


def _phi(x):
    return jnp.where(x > 0, x + 1.0, jnp.exp(x))


def _qkv_kernel(x_ref, wq_ref, wk_ref, wv_ref,
                q_ref, k_ref, v_ref, qp_ref, kp_ref):
    x = x_ref[0]
    q = jnp.dot(x, wq_ref[:, :], preferred_element_type=jnp.float32)
    k = jnp.dot(x, wk_ref[:, :], preferred_element_type=jnp.float32)
    v = jnp.dot(x, wv_ref[:, :], preferred_element_type=jnp.float32)
    scale = 1.0 / BS
    qp = jnp.stack([jnp.sum(q[j * BS:(j + 1) * BS, :], axis=0) * scale
                    for j in range(NPOOL)], axis=0)
    kp = jnp.stack([jnp.sum(k[j * BS:(j + 1) * BS, :], axis=0) * scale
                    for j in range(NPOOL)], axis=0)
    for h in range(H):
        sl = slice(h * DH, (h + 1) * DH)
        q_ref[0, h] = q[:, sl].astype(jnp.bfloat16)
        k_ref[0, h] = k[:, sl].astype(jnp.bfloat16)
        v_ref[0, h] = v[:, sl].astype(jnp.bfloat16)
        qp_ref[0, h] = qp[:, sl]
        kp_ref[0, h] = kp[:, sl]


def _qkv(hidden, Wq, Wk, Wv):
    grid = (B, L // LB)
    w_spec = pl.BlockSpec((DM, DM), lambda b, i: (0, 0))
    return pl.pallas_call(
        _qkv_kernel,
        grid=grid,
        in_specs=[
            pl.BlockSpec((1, LB, DM), lambda b, i: (b, i, 0)),
            w_spec, w_spec, w_spec,
        ],
        out_specs=[
            pl.BlockSpec((1, H, LB, DH), lambda b, i: (b, 0, i, 0)),
            pl.BlockSpec((1, H, LB, DH), lambda b, i: (b, 0, i, 0)),
            pl.BlockSpec((1, H, LB, DH), lambda b, i: (b, 0, i, 0)),
            pl.BlockSpec((1, H, NPOOL, DH), lambda b, i: (b, 0, i, 0)),
            pl.BlockSpec((1, H, NPOOL, DH), lambda b, i: (b, 0, i, 0)),
        ],
        out_shape=[
            jax.ShapeDtypeStruct((B, H, L, DH), jnp.bfloat16),
            jax.ShapeDtypeStruct((B, H, L, DH), jnp.bfloat16),
            jax.ShapeDtypeStruct((B, H, L, DH), jnp.bfloat16),
            jax.ShapeDtypeStruct((B, H, NB, DH), jnp.float32),
            jax.ShapeDtypeStruct((B, H, NB, DH), jnp.float32),
        ],
        interpret=_INTERPRET,
    )(hidden, Wq, Wk, Wv)


def _scores_kernel(qp_ref, kp_ref, rq_ref, rk_ref, s_ref):
    qp = jnp.dot(qp_ref[0, 0], rq_ref[:, :], preferred_element_type=jnp.float32)
    kp = jnp.dot(kp_ref[0, 0], rk_ref[:, :], preferred_element_type=jnp.float32)
    s_ref[0, 0] = lax.dot_general(qp, kp, (((1,), (1,)), ((), ())),
                                  preferred_element_type=jnp.float32) * (1.0 / 8.0)


def _scores(q_pool, k_pool, Rq, Rk):
    grid = (B, H)
    r_spec = pl.BlockSpec((DH, DH), lambda b, h: (0, 0))
    return pl.pallas_call(
        _scores_kernel,
        grid=grid,
        in_specs=[
            pl.BlockSpec((1, 1, NB, DH), lambda b, h: (b, h, 0, 0)),
            pl.BlockSpec((1, 1, NB, DH), lambda b, h: (b, h, 0, 0)),
            r_spec, r_spec,
        ],
        out_specs=pl.BlockSpec((1, 1, NB, NB), lambda b, h: (b, h, 0, 0)),
        out_shape=jax.ShapeDtypeStruct((B, H, NB, NB), jnp.float32),
        interpret=_INTERPRET,
    )(q_pool, k_pool, Rq, Rk)


NROWS = B * H * NB
NWORK = 32
RW = NROWS // NWORK


def _topk_sc_kernel(s_hbm, sel_hbm, s_v, o_v, kbuf, vbuf):
    from jax.experimental.pallas import tpu_sc as plsc
    wid = lax.axis_index("s") * 2 + lax.axis_index("c")
    base = wid * RW
    pltpu.sync_copy(s_hbm.at[pl.ds(base, RW)], s_v)

    idx16 = lax.iota(jnp.int32, 16)

    def row(i, _):
        for pair, dst in ((0, 0), (1, 8)):
            for c, off in ((2 * pair, 16), (2 * pair + 1, 24)):
                ks, vs = plsc.sort_key_val(
                    s_v[i, pl.ds(c * 16, 16)], idx16 + 16 * c,
                    descending=True)
                kbuf[pl.ds(off, 16)] = ks
                vbuf[pl.ds(off, 16)] = vs
            km, vm = plsc.sort_key_val(
                kbuf[pl.ds(16, 16)], vbuf[pl.ds(16, 16)], descending=True)
            kbuf[pl.ds(dst, 16)] = km
            vbuf[pl.ds(dst, 16)] = vm
        _, vf = plsc.sort_key_val(
            kbuf[pl.ds(0, 16)], vbuf[pl.ds(0, 16)], descending=True)
        o_v[i, :] = vf
        return 0

    lax.fori_loop(0, RW, row, 0)
    pltpu.sync_copy(o_v, sel_hbm.at[pl.ds(base, RW)])


def _topk_sc(scores_flat):
    import functools as _ft
    from jax.experimental.pallas import tpu_sc as plsc
    mesh = plsc.VectorSubcoreMesh(core_axis_name="c", subcore_axis_name="s")
    kern = _ft.partial(
        pl.kernel,
        mesh=mesh,
        out_type=jax.ShapeDtypeStruct((NROWS, 16), jnp.int32),
        scratch_types=[
            pltpu.VMEM((RW, NB), jnp.float32),
            pltpu.VMEM((RW, 16), jnp.int32),
            pltpu.VMEM((40,), jnp.float32),
            pltpu.VMEM((40,), jnp.int32),
        ],
        compiler_params=pltpu.CompilerParams(needs_layout_passes=False),
    )(_topk_sc_kernel)
    return kern(scores_flat)


def _linstate_kernel(k_ref, v_ref, kv_ref, z_ref):
    pk = _phi(k_ref[0, 0].astype(jnp.float32))
    kv_ref[0, 0] = lax.dot_general(
        pk.astype(jnp.bfloat16), v_ref[0, 0], (((0,), (0,)), ((), ())),
        preferred_element_type=jnp.float32)
    z_ref[0, 0] = jnp.sum(pk, axis=0, keepdims=True)


def _linstate(k, v):
    grid = (B, H)
    return pl.pallas_call(
        _linstate_kernel,
        grid=grid,
        in_specs=[
            pl.BlockSpec((1, 1, L, DH), lambda b, h: (b, h, 0, 0)),
            pl.BlockSpec((1, 1, L, DH), lambda b, h: (b, h, 0, 0)),
        ],
        out_specs=[
            pl.BlockSpec((1, 1, DH, DH), lambda b, h: (b, h, 0, 0)),
            pl.BlockSpec((1, 1, 1, DH), lambda b, h: (b, h, 0, 0)),
        ],
        out_shape=[
            jax.ShapeDtypeStruct((B, H, DH, DH), jnp.float32),
            jax.ShapeDtypeStruct((B, H, 1, DH), jnp.float32),
        ],
        interpret=_INTERPRET,
    )(k, v)


GQ = 4


def _attn_kernel(sel_ref, alpha_ref, q_ref, k_ref, v_ref, kv_ref, z_ref,
                 o_ref):
    b = pl.program_id(0)
    h = pl.program_id(1)
    qi = pl.program_id(2)

    a = alpha_ref[h]
    kv = kv_ref[0, 0]
    z = z_ref[0, 0]
    for g in range(GQ):
        q_blk = q_ref[0, 0, g * BS:(g + 1) * BS, :]
        base = ((b * H + h) * NB + qi * GQ + g) * KK
        k_rows = []
        v_rows = []
        for j in range(KK):
            idx = sel_ref[base + j]
            k_rows.append(k_ref[0, 0, pl.ds(idx * BS, BS), :])
            v_rows.append(v_ref[0, 0, pl.ds(idx * BS, BS), :])
        k_cat = jnp.concatenate(k_rows, axis=0)
        v_cat = jnp.concatenate(v_rows, axis=0)
        s = lax.dot_general(q_blk, k_cat, (((1,), (1,)), ((), ())),
                            preferred_element_type=jnp.float32)
        m = jnp.max(s, axis=1, keepdims=True)
        e = jnp.exp(s - m)
        den_s = jnp.sum(e, axis=1, keepdims=True)
        o_s = jnp.dot(e.astype(jnp.bfloat16), v_cat,
                      preferred_element_type=jnp.float32) / den_s

        pq = _phi(q_blk.astype(jnp.float32))
        num = jnp.dot(pq, kv, preferred_element_type=jnp.float32)
        den_l = jnp.sum(pq * z, axis=1, keepdims=True) + 1e-6
        o_l = num / den_l

        o_ref[0, 0, g * BS:(g + 1) * BS, :] = (
            a * o_s + (1.0 - a) * o_l).astype(jnp.bfloat16)


def _attention(q, k, v, kv, z, sel_flat, alpha):
    grid = (B, H, NB // GQ)
    grid_spec = pltpu.PrefetchScalarGridSpec(
        num_scalar_prefetch=2,
        grid=grid,
        in_specs=[
            pl.BlockSpec((1, 1, GQ * BS, DH), lambda b, h, qi, *_: (b, h, qi, 0)),
            pl.BlockSpec((1, 1, L, DH), lambda b, h, qi, *_: (b, h, 0, 0)),
            pl.BlockSpec((1, 1, L, DH), lambda b, h, qi, *_: (b, h, 0, 0)),
            pl.BlockSpec((1, 1, DH, DH), lambda b, h, qi, *_: (b, h, 0, 0)),
            pl.BlockSpec((1, 1, 1, DH), lambda b, h, qi, *_: (b, h, 0, 0)),
        ],
        out_specs=pl.BlockSpec((1, 1, GQ * BS, DH),
                               lambda b, h, qi, *_: (b, h, qi, 0)),
    )
    return pl.pallas_call(
        _attn_kernel,
        grid_spec=grid_spec,
        out_shape=jax.ShapeDtypeStruct((B, H, L, DH), jnp.bfloat16),
        interpret=_INTERPRET,
    )(sel_flat, alpha, q, k, v, kv, z)


def _proj_kernel(x_ref, w_ref, o_ref):
    x = jnp.concatenate([x_ref[0, h] for h in range(H)], axis=1)
    o_ref[0] = jnp.dot(x, w_ref[:, :], preferred_element_type=jnp.float32)


def _out_proj(x, Wo):
    grid = (B, L // LB)
    return pl.pallas_call(
        _proj_kernel,
        grid=grid,
        in_specs=[
            pl.BlockSpec((1, H, LB, DH), lambda b, i: (b, 0, i, 0)),
            pl.BlockSpec((DM, DM), lambda b, i: (0, 0)),
        ],
        out_specs=pl.BlockSpec((1, LB, DM), lambda b, i: (b, i, 0)),
        out_shape=jax.ShapeDtypeStruct((B, L, DM), jnp.float32),
        interpret=_INTERPRET,
    )(x, Wo)


@jax.jit
def kernel(hidden_states, Wq, Wk, Wv, Wo, Rq, Rk, alpha_logits):
    q, k, v, q_pool, k_pool = _qkv(hidden_states, Wq, Wk, Wv)
    scores = _scores(q_pool, k_pool, Rq, Rk)
    sel_pad = _topk_sc(scores.reshape(NROWS, NB))
    sel_flat = sel_pad[:, :KK].reshape(-1)
    alpha = jax.nn.sigmoid(alpha_logits).reshape(H)
    kv, z = _linstate(k, v)
    out_attn = _attention(q, k, v, kv, z, sel_flat, alpha)
    return _out_proj(out_attn, Wo.astype(jnp.bfloat16))


# speedup vs baseline: 1.5849x; 1.0884x over previous
import functools

import jax
import jax.numpy as jnp
from jax import lax
from jax.experimental import pallas as pl
from jax.experimental.pallas import tpu as pltpu

B, L, DM, H, BS = 2, 4096, 1024, 16, 64
DH = DM // H
NB = L // BS
KK = 8
LB = 512
NPOOL = LB // BS

_INTERPRET = False


def _phi(x):
    return jnp.where(x > 0, x + 1.0, jnp.exp(x))


def _qkv_kernel(x_ref, wq_ref, wk_ref, wv_ref,
                q_ref, k_ref, v_ref, qp_ref, kp_ref):
    x = x_ref[0]
    q = jnp.dot(x, wq_ref[:, :], preferred_element_type=jnp.float32)
    k = jnp.dot(x, wk_ref[:, :], preferred_element_type=jnp.float32)
    v = jnp.dot(x, wv_ref[:, :], preferred_element_type=jnp.float32)
    scale = 1.0 / BS
    qp = jnp.stack([jnp.sum(q[j * BS:(j + 1) * BS, :], axis=0) * scale
                    for j in range(NPOOL)], axis=0)
    kp = jnp.stack([jnp.sum(k[j * BS:(j + 1) * BS, :], axis=0) * scale
                    for j in range(NPOOL)], axis=0)
    for h in range(H):
        sl = slice(h * DH, (h + 1) * DH)
        q_ref[0, h] = q[:, sl].astype(jnp.bfloat16)
        k_ref[0, h] = k[:, sl].astype(jnp.bfloat16)
        v_ref[0, h] = v[:, sl].astype(jnp.bfloat16)
        qp_ref[0, h] = qp[:, sl]
        kp_ref[0, h] = kp[:, sl]


def _qkv(hidden, Wq, Wk, Wv):
    grid = (B, L // LB)
    w_spec = pl.BlockSpec((DM, DM), lambda b, i: (0, 0))
    return pl.pallas_call(
        _qkv_kernel,
        grid=grid,
        in_specs=[
            pl.BlockSpec((1, LB, DM), lambda b, i: (b, i, 0)),
            w_spec, w_spec, w_spec,
        ],
        out_specs=[
            pl.BlockSpec((1, H, LB, DH), lambda b, i: (b, 0, i, 0)),
            pl.BlockSpec((1, H, LB, DH), lambda b, i: (b, 0, i, 0)),
            pl.BlockSpec((1, H, LB, DH), lambda b, i: (b, 0, i, 0)),
            pl.BlockSpec((1, H, NPOOL, DH), lambda b, i: (b, 0, i, 0)),
            pl.BlockSpec((1, H, NPOOL, DH), lambda b, i: (b, 0, i, 0)),
        ],
        out_shape=[
            jax.ShapeDtypeStruct((B, H, L, DH), jnp.bfloat16),
            jax.ShapeDtypeStruct((B, H, L, DH), jnp.bfloat16),
            jax.ShapeDtypeStruct((B, H, L, DH), jnp.bfloat16),
            jax.ShapeDtypeStruct((B, H, NB, DH), jnp.float32),
            jax.ShapeDtypeStruct((B, H, NB, DH), jnp.float32),
        ],
        interpret=_INTERPRET,
    )(hidden, Wq, Wk, Wv)


def _scores_kernel(qp_ref, kp_ref, rq_ref, rk_ref, s_ref):
    qp = jnp.dot(qp_ref[0, 0], rq_ref[:, :], preferred_element_type=jnp.float32)
    kp = jnp.dot(kp_ref[0, 0], rk_ref[:, :], preferred_element_type=jnp.float32)
    s_ref[0, 0] = lax.dot_general(qp, kp, (((1,), (1,)), ((), ())),
                                  preferred_element_type=jnp.float32) * (1.0 / 8.0)


def _scores(q_pool, k_pool, Rq, Rk):
    grid = (B, H)
    r_spec = pl.BlockSpec((DH, DH), lambda b, h: (0, 0))
    return pl.pallas_call(
        _scores_kernel,
        grid=grid,
        in_specs=[
            pl.BlockSpec((1, 1, NB, DH), lambda b, h: (b, h, 0, 0)),
            pl.BlockSpec((1, 1, NB, DH), lambda b, h: (b, h, 0, 0)),
            r_spec, r_spec,
        ],
        out_specs=pl.BlockSpec((1, 1, NB, NB), lambda b, h: (b, h, 0, 0)),
        out_shape=jax.ShapeDtypeStruct((B, H, NB, NB), jnp.float32),
        interpret=_INTERPRET,
    )(q_pool, k_pool, Rq, Rk)


NROWS = B * H * NB
NWORK = 32
RW = NROWS // NWORK


def _topk_sc_kernel(s_hbm, sel_hbm, s_v, o_v, kbuf, vbuf):
    from jax.experimental.pallas import tpu_sc as plsc
    wid = lax.axis_index("s") * 2 + lax.axis_index("c")
    base = wid * RW
    pltpu.sync_copy(s_hbm.at[pl.ds(base, RW)], s_v)

    idx16 = lax.iota(jnp.int32, 16)

    def row(i, _):
        for pair, dst in ((0, 0), (1, 8)):
            for c, off in ((2 * pair, 16), (2 * pair + 1, 24)):
                ks, vs = plsc.sort_key_val(
                    s_v[i, pl.ds(c * 16, 16)], idx16 + 16 * c,
                    descending=True)
                kbuf[pl.ds(off, 16)] = ks
                vbuf[pl.ds(off, 16)] = vs
            km, vm = plsc.sort_key_val(
                kbuf[pl.ds(16, 16)], vbuf[pl.ds(16, 16)], descending=True)
            kbuf[pl.ds(dst, 16)] = km
            vbuf[pl.ds(dst, 16)] = vm
        _, vf = plsc.sort_key_val(
            kbuf[pl.ds(0, 16)], vbuf[pl.ds(0, 16)], descending=True)
        o_v[i, :] = vf
        return 0

    lax.fori_loop(0, RW, row, 0)
    pltpu.sync_copy(o_v, sel_hbm.at[pl.ds(base, RW)])


def _topk_sc(scores_flat):
    import functools as _ft
    from jax.experimental.pallas import tpu_sc as plsc
    mesh = plsc.VectorSubcoreMesh(core_axis_name="c", subcore_axis_name="s")
    kern = _ft.partial(
        pl.kernel,
        mesh=mesh,
        out_type=jax.ShapeDtypeStruct((NROWS, 16), jnp.int32),
        scratch_types=[
            pltpu.VMEM((RW, NB), jnp.float32),
            pltpu.VMEM((RW, 16), jnp.int32),
            pltpu.VMEM((40,), jnp.float32),
            pltpu.VMEM((40,), jnp.int32),
        ],
        compiler_params=pltpu.CompilerParams(needs_layout_passes=False),
    )(_topk_sc_kernel)
    return kern(scores_flat)


def _linstate_kernel(k_ref, v_ref, kv_ref, z_ref):
    pk = _phi(k_ref[0, 0].astype(jnp.float32))
    kv_ref[0, 0] = lax.dot_general(
        pk.astype(jnp.bfloat16), v_ref[0, 0], (((0,), (0,)), ((), ())),
        preferred_element_type=jnp.float32)
    z_ref[0, 0] = jnp.sum(pk, axis=0, keepdims=True)


def _linstate(k, v):
    grid = (B, H)
    return pl.pallas_call(
        _linstate_kernel,
        grid=grid,
        in_specs=[
            pl.BlockSpec((1, 1, L, DH), lambda b, h: (b, h, 0, 0)),
            pl.BlockSpec((1, 1, L, DH), lambda b, h: (b, h, 0, 0)),
        ],
        out_specs=[
            pl.BlockSpec((1, 1, DH, DH), lambda b, h: (b, h, 0, 0)),
            pl.BlockSpec((1, 1, 1, DH), lambda b, h: (b, h, 0, 0)),
        ],
        out_shape=[
            jax.ShapeDtypeStruct((B, H, DH, DH), jnp.float32),
            jax.ShapeDtypeStruct((B, H, 1, DH), jnp.float32),
        ],
        interpret=_INTERPRET,
    )(k, v)


GQ = 8


def _attn_kernel(sel_ref, alpha_ref, q_ref, k_ref, v_ref, kv_ref, z_ref,
                 o_ref):
    b = pl.program_id(0)
    h = pl.program_id(1)
    qi = pl.program_id(2)

    a = alpha_ref[h]
    kv = kv_ref[0, 0]
    z = z_ref[0, 0]
    for g in range(GQ):
        q_blk = q_ref[0, 0, g * BS:(g + 1) * BS, :]
        base = ((b * H + h) * NB + qi * GQ + g) * KK
        k_rows = []
        v_rows = []
        for j in range(KK):
            idx = sel_ref[base + j]
            k_rows.append(k_ref[0, 0, pl.ds(idx * BS, BS), :])
            v_rows.append(v_ref[0, 0, pl.ds(idx * BS, BS), :])
        k_cat = jnp.concatenate(k_rows, axis=0)
        v_cat = jnp.concatenate(v_rows, axis=0)
        s = lax.dot_general(q_blk, k_cat, (((1,), (1,)), ((), ())),
                            preferred_element_type=jnp.float32)
        m = jnp.max(s, axis=1, keepdims=True)
        e = jnp.exp(s - m)
        den_s = jnp.sum(e, axis=1, keepdims=True)
        o_s = jnp.dot(e.astype(jnp.bfloat16), v_cat,
                      preferred_element_type=jnp.float32) / den_s

        pq = _phi(q_blk.astype(jnp.float32))
        num = jnp.dot(pq, kv, preferred_element_type=jnp.float32)
        den_l = jnp.sum(pq * z, axis=1, keepdims=True) + 1e-6
        o_l = num / den_l

        o_ref[0, 0, g * BS:(g + 1) * BS, :] = (
            a * o_s + (1.0 - a) * o_l).astype(jnp.bfloat16)


def _attention(q, k, v, kv, z, sel_flat, alpha):
    grid = (B, H, NB // GQ)
    grid_spec = pltpu.PrefetchScalarGridSpec(
        num_scalar_prefetch=2,
        grid=grid,
        in_specs=[
            pl.BlockSpec((1, 1, GQ * BS, DH), lambda b, h, qi, *_: (b, h, qi, 0)),
            pl.BlockSpec((1, 1, L, DH), lambda b, h, qi, *_: (b, h, 0, 0)),
            pl.BlockSpec((1, 1, L, DH), lambda b, h, qi, *_: (b, h, 0, 0)),
            pl.BlockSpec((1, 1, DH, DH), lambda b, h, qi, *_: (b, h, 0, 0)),
            pl.BlockSpec((1, 1, 1, DH), lambda b, h, qi, *_: (b, h, 0, 0)),
        ],
        out_specs=pl.BlockSpec((1, 1, GQ * BS, DH),
                               lambda b, h, qi, *_: (b, h, qi, 0)),
    )
    return pl.pallas_call(
        _attn_kernel,
        grid_spec=grid_spec,
        out_shape=jax.ShapeDtypeStruct((B, H, L, DH), jnp.bfloat16),
        interpret=_INTERPRET,
    )(sel_flat, alpha, q, k, v, kv, z)


def _proj_kernel(x_ref, w_ref, o_ref):
    x = jnp.concatenate([x_ref[0, h] for h in range(H)], axis=1)
    o_ref[0] = jnp.dot(x, w_ref[:, :], preferred_element_type=jnp.float32)


def _out_proj(x, Wo):
    grid = (B, L // LB)
    return pl.pallas_call(
        _proj_kernel,
        grid=grid,
        in_specs=[
            pl.BlockSpec((1, H, LB, DH), lambda b, i: (b, 0, i, 0)),
            pl.BlockSpec((DM, DM), lambda b, i: (0, 0)),
        ],
        out_specs=pl.BlockSpec((1, LB, DM), lambda b, i: (b, i, 0)),
        out_shape=jax.ShapeDtypeStruct((B, L, DM), jnp.float32),
        interpret=_INTERPRET,
    )(x, Wo)


@jax.jit
def kernel(hidden_states, Wq, Wk, Wv, Wo, Rq, Rk, alpha_logits):
    q, k, v, q_pool, k_pool = _qkv(hidden_states, Wq, Wk, Wv)
    scores = _scores(q_pool, k_pool, Rq, Rk)
    sel_pad = _topk_sc(scores.reshape(NROWS, NB))
    sel_flat = sel_pad[:, :KK].reshape(-1)
    alpha = jax.nn.sigmoid(alpha_logits).reshape(H)
    kv, z = _linstate(k, v)
    out_attn = _attention(q, k, v, kv, z, sel_flat, alpha)
    return _out_proj(out_attn, Wo.astype(jnp.bfloat16))


# speedup vs baseline: 1.6249x; 1.0252x over previous
import functools

import jax
import jax.numpy as jnp
from jax import lax
from jax.experimental import pallas as pl
from jax.experimental.pallas import tpu as pltpu

B, L, DM, H, BS = 2, 4096, 1024, 16, 64
DH = DM // H
NB = L // BS
KK = 8
LB = 512
NPOOL = LB // BS

_INTERPRET = False


def _phi(x):
    return jnp.where(x > 0, x + 1.0, jnp.exp(x))


def _qkv_kernel(x_ref, wq_ref, wk_ref, wv_ref,
                q_ref, k_ref, v_ref, qp_ref, kp_ref):
    x = x_ref[0]
    q = jnp.dot(x, wq_ref[:, :], preferred_element_type=jnp.float32)
    k = jnp.dot(x, wk_ref[:, :], preferred_element_type=jnp.float32)
    v = jnp.dot(x, wv_ref[:, :], preferred_element_type=jnp.float32)
    scale = 1.0 / BS
    qp = jnp.stack([jnp.sum(q[j * BS:(j + 1) * BS, :], axis=0) * scale
                    for j in range(NPOOL)], axis=0)
    kp = jnp.stack([jnp.sum(k[j * BS:(j + 1) * BS, :], axis=0) * scale
                    for j in range(NPOOL)], axis=0)
    for h in range(H):
        sl = slice(h * DH, (h + 1) * DH)
        q_ref[0, h] = q[:, sl].astype(jnp.bfloat16)
        k_ref[0, h] = k[:, sl].astype(jnp.bfloat16)
        v_ref[0, h] = v[:, sl].astype(jnp.bfloat16)
        qp_ref[0, h] = qp[:, sl]
        kp_ref[0, h] = kp[:, sl]


def _qkv(hidden, Wq, Wk, Wv):
    grid = (B, L // LB)
    w_spec = pl.BlockSpec((DM, DM), lambda b, i: (0, 0))
    return pl.pallas_call(
        _qkv_kernel,
        grid=grid,
        in_specs=[
            pl.BlockSpec((1, LB, DM), lambda b, i: (b, i, 0)),
            w_spec, w_spec, w_spec,
        ],
        out_specs=[
            pl.BlockSpec((1, H, LB, DH), lambda b, i: (b, 0, i, 0)),
            pl.BlockSpec((1, H, LB, DH), lambda b, i: (b, 0, i, 0)),
            pl.BlockSpec((1, H, LB, DH), lambda b, i: (b, 0, i, 0)),
            pl.BlockSpec((1, H, NPOOL, DH), lambda b, i: (b, 0, i, 0)),
            pl.BlockSpec((1, H, NPOOL, DH), lambda b, i: (b, 0, i, 0)),
        ],
        out_shape=[
            jax.ShapeDtypeStruct((B, H, L, DH), jnp.bfloat16),
            jax.ShapeDtypeStruct((B, H, L, DH), jnp.bfloat16),
            jax.ShapeDtypeStruct((B, H, L, DH), jnp.bfloat16),
            jax.ShapeDtypeStruct((B, H, NB, DH), jnp.float32),
            jax.ShapeDtypeStruct((B, H, NB, DH), jnp.float32),
        ],
        interpret=_INTERPRET,
    )(hidden, Wq, Wk, Wv)


def _scores_kernel(qp_ref, kp_ref, rq_ref, rk_ref, s_ref):
    qp = jnp.dot(qp_ref[0, 0], rq_ref[:, :], preferred_element_type=jnp.float32)
    kp = jnp.dot(kp_ref[0, 0], rk_ref[:, :], preferred_element_type=jnp.float32)
    s_ref[0, 0] = lax.dot_general(qp, kp, (((1,), (1,)), ((), ())),
                                  preferred_element_type=jnp.float32) * (1.0 / 8.0)


def _scores(q_pool, k_pool, Rq, Rk):
    grid = (B, H)
    r_spec = pl.BlockSpec((DH, DH), lambda b, h: (0, 0))
    return pl.pallas_call(
        _scores_kernel,
        grid=grid,
        in_specs=[
            pl.BlockSpec((1, 1, NB, DH), lambda b, h: (b, h, 0, 0)),
            pl.BlockSpec((1, 1, NB, DH), lambda b, h: (b, h, 0, 0)),
            r_spec, r_spec,
        ],
        out_specs=pl.BlockSpec((1, 1, NB, NB), lambda b, h: (b, h, 0, 0)),
        out_shape=jax.ShapeDtypeStruct((B, H, NB, NB), jnp.float32),
        interpret=_INTERPRET,
    )(q_pool, k_pool, Rq, Rk)


NROWS = B * H * NB
NWORK = 32
RW = NROWS // NWORK


def _topk_sc_kernel(s_hbm, sel_hbm, s_v, o_v, kbuf, vbuf):
    from jax.experimental.pallas import tpu_sc as plsc
    wid = lax.axis_index("s") * 2 + lax.axis_index("c")
    base = wid * RW
    pltpu.sync_copy(s_hbm.at[pl.ds(base, RW)], s_v)

    idx16 = lax.iota(jnp.int32, 16)

    def row(i, _):
        for pair, dst in ((0, 0), (1, 8)):
            for c, off in ((2 * pair, 16), (2 * pair + 1, 24)):
                ks, vs = plsc.sort_key_val(
                    s_v[i, pl.ds(c * 16, 16)], idx16 + 16 * c,
                    descending=True)
                kbuf[pl.ds(off, 16)] = ks
                vbuf[pl.ds(off, 16)] = vs
            km, vm = plsc.sort_key_val(
                kbuf[pl.ds(16, 16)], vbuf[pl.ds(16, 16)], descending=True)
            kbuf[pl.ds(dst, 16)] = km
            vbuf[pl.ds(dst, 16)] = vm
        _, vf = plsc.sort_key_val(
            kbuf[pl.ds(0, 16)], vbuf[pl.ds(0, 16)], descending=True)
        o_v[i, :] = vf
        return 0

    lax.fori_loop(0, RW, row, 0)
    pltpu.sync_copy(o_v, sel_hbm.at[pl.ds(base, RW)])


def _topk_sc(scores_flat):
    import functools as _ft
    from jax.experimental.pallas import tpu_sc as plsc
    mesh = plsc.VectorSubcoreMesh(core_axis_name="c", subcore_axis_name="s")
    kern = _ft.partial(
        pl.kernel,
        mesh=mesh,
        out_type=jax.ShapeDtypeStruct((NROWS, 16), jnp.int32),
        scratch_types=[
            pltpu.VMEM((RW, NB), jnp.float32),
            pltpu.VMEM((RW, 16), jnp.int32),
            pltpu.VMEM((40,), jnp.float32),
            pltpu.VMEM((40,), jnp.int32),
        ],
        compiler_params=pltpu.CompilerParams(needs_layout_passes=False),
    )(_topk_sc_kernel)
    return kern(scores_flat)


def _linstate_kernel(k_ref, v_ref, kv_ref, z_ref):
    pk = _phi(k_ref[0, 0].astype(jnp.float32))
    kv_ref[0, 0] = lax.dot_general(
        pk.astype(jnp.bfloat16), v_ref[0, 0], (((0,), (0,)), ((), ())),
        preferred_element_type=jnp.float32)
    z_ref[0, 0] = jnp.sum(pk, axis=0, keepdims=True)


def _linstate(k, v):
    grid = (B, H)
    return pl.pallas_call(
        _linstate_kernel,
        grid=grid,
        in_specs=[
            pl.BlockSpec((1, 1, L, DH), lambda b, h: (b, h, 0, 0)),
            pl.BlockSpec((1, 1, L, DH), lambda b, h: (b, h, 0, 0)),
        ],
        out_specs=[
            pl.BlockSpec((1, 1, DH, DH), lambda b, h: (b, h, 0, 0)),
            pl.BlockSpec((1, 1, 1, DH), lambda b, h: (b, h, 0, 0)),
        ],
        out_shape=[
            jax.ShapeDtypeStruct((B, H, DH, DH), jnp.float32),
            jax.ShapeDtypeStruct((B, H, 1, DH), jnp.float32),
        ],
        interpret=_INTERPRET,
    )(k, v)


GQ = 16


def _attn_kernel(sel_ref, alpha_ref, q_ref, k_ref, v_ref, kv_ref, z_ref,
                 o_ref):
    b = pl.program_id(0)
    h = pl.program_id(1)
    qi = pl.program_id(2)

    a = alpha_ref[h]
    kv = kv_ref[0, 0]
    z = z_ref[0, 0]
    for g in range(GQ):
        q_blk = q_ref[0, 0, g * BS:(g + 1) * BS, :]
        base = ((b * H + h) * NB + qi * GQ + g) * KK
        k_rows = []
        v_rows = []
        for j in range(KK):
            idx = sel_ref[base + j]
            k_rows.append(k_ref[0, 0, pl.ds(idx * BS, BS), :])
            v_rows.append(v_ref[0, 0, pl.ds(idx * BS, BS), :])
        k_cat = jnp.concatenate(k_rows, axis=0)
        v_cat = jnp.concatenate(v_rows, axis=0)
        s = lax.dot_general(q_blk, k_cat, (((1,), (1,)), ((), ())),
                            preferred_element_type=jnp.float32)
        m = jnp.max(s, axis=1, keepdims=True)
        e = jnp.exp(s - m)
        den_s = jnp.sum(e, axis=1, keepdims=True)
        o_s = jnp.dot(e.astype(jnp.bfloat16), v_cat,
                      preferred_element_type=jnp.float32) / den_s

        pq = _phi(q_blk.astype(jnp.float32))
        num = jnp.dot(pq, kv, preferred_element_type=jnp.float32)
        den_l = jnp.sum(pq * z, axis=1, keepdims=True) + 1e-6
        o_l = num / den_l

        o_ref[0, 0, g * BS:(g + 1) * BS, :] = (
            a * o_s + (1.0 - a) * o_l).astype(jnp.bfloat16)


def _attention(q, k, v, kv, z, sel_flat, alpha):
    grid = (B, H, NB // GQ)
    grid_spec = pltpu.PrefetchScalarGridSpec(
        num_scalar_prefetch=2,
        grid=grid,
        in_specs=[
            pl.BlockSpec((1, 1, GQ * BS, DH), lambda b, h, qi, *_: (b, h, qi, 0)),
            pl.BlockSpec((1, 1, L, DH), lambda b, h, qi, *_: (b, h, 0, 0)),
            pl.BlockSpec((1, 1, L, DH), lambda b, h, qi, *_: (b, h, 0, 0)),
            pl.BlockSpec((1, 1, DH, DH), lambda b, h, qi, *_: (b, h, 0, 0)),
            pl.BlockSpec((1, 1, 1, DH), lambda b, h, qi, *_: (b, h, 0, 0)),
        ],
        out_specs=pl.BlockSpec((1, 1, GQ * BS, DH),
                               lambda b, h, qi, *_: (b, h, qi, 0)),
    )
    return pl.pallas_call(
        _attn_kernel,
        grid_spec=grid_spec,
        out_shape=jax.ShapeDtypeStruct((B, H, L, DH), jnp.bfloat16),
        interpret=_INTERPRET,
    )(sel_flat, alpha, q, k, v, kv, z)


def _proj_kernel(x_ref, w_ref, o_ref):
    x = jnp.concatenate([x_ref[0, h] for h in range(H)], axis=1)
    o_ref[0] = jnp.dot(x, w_ref[:, :], preferred_element_type=jnp.float32)


def _out_proj(x, Wo):
    grid = (B, L // LB)
    return pl.pallas_call(
        _proj_kernel,
        grid=grid,
        in_specs=[
            pl.BlockSpec((1, H, LB, DH), lambda b, i: (b, 0, i, 0)),
            pl.BlockSpec((DM, DM), lambda b, i: (0, 0)),
        ],
        out_specs=pl.BlockSpec((1, LB, DM), lambda b, i: (b, i, 0)),
        out_shape=jax.ShapeDtypeStruct((B, L, DM), jnp.float32),
        interpret=_INTERPRET,
    )(x, Wo)


@jax.jit
def kernel(hidden_states, Wq, Wk, Wv, Wo, Rq, Rk, alpha_logits):
    q, k, v, q_pool, k_pool = _qkv(hidden_states, Wq, Wk, Wv)
    scores = _scores(q_pool, k_pool, Rq, Rk)
    sel_pad = _topk_sc(scores.reshape(NROWS, NB))
    sel_flat = sel_pad[:, :KK].reshape(-1)
    alpha = jax.nn.sigmoid(alpha_logits).reshape(H)
    kv, z = _linstate(k, v)
    out_attn = _attention(q, k, v, kv, z, sel_flat, alpha)
    return _out_proj(out_attn, Wo.astype(jnp.bfloat16))


# speedup vs baseline: 1.6507x; 1.0159x over previous
import functools

import jax
import jax.numpy as jnp
from jax import lax
from jax.experimental import pallas as pl
from jax.experimental.pallas import tpu as pltpu

B, L, DM, H, BS = 2, 4096, 1024, 16, 64
DH = DM // H
NB = L // BS
KK = 8
LB = 512
NPOOL = LB // BS

_INTERPRET = False


def _phi(x):
    return jnp.where(x > 0, x + 1.0, jnp.exp(x))


def _qkv_kernel(x_ref, wq_ref, wk_ref, wv_ref,
                q_ref, k_ref, v_ref, qp_ref, kp_ref):
    x = x_ref[0]
    q = jnp.dot(x, wq_ref[:, :], preferred_element_type=jnp.float32)
    k = jnp.dot(x, wk_ref[:, :], preferred_element_type=jnp.float32)
    v = jnp.dot(x, wv_ref[:, :], preferred_element_type=jnp.float32)
    scale = 1.0 / BS
    qp = jnp.stack([jnp.sum(q[j * BS:(j + 1) * BS, :], axis=0) * scale
                    for j in range(NPOOL)], axis=0)
    kp = jnp.stack([jnp.sum(k[j * BS:(j + 1) * BS, :], axis=0) * scale
                    for j in range(NPOOL)], axis=0)
    for h in range(H):
        sl = slice(h * DH, (h + 1) * DH)
        q_ref[0, h] = q[:, sl].astype(jnp.bfloat16)
        k_ref[0, h] = k[:, sl].astype(jnp.bfloat16)
        v_ref[0, h] = v[:, sl].astype(jnp.bfloat16)
        qp_ref[0, h] = qp[:, sl]
        kp_ref[0, h] = kp[:, sl]


def _qkv(hidden, Wq, Wk, Wv):
    grid = (B, L // LB)
    w_spec = pl.BlockSpec((DM, DM), lambda b, i: (0, 0))
    return pl.pallas_call(
        _qkv_kernel,
        grid=grid,
        in_specs=[
            pl.BlockSpec((1, LB, DM), lambda b, i: (b, i, 0)),
            w_spec, w_spec, w_spec,
        ],
        out_specs=[
            pl.BlockSpec((1, H, LB, DH), lambda b, i: (b, 0, i, 0)),
            pl.BlockSpec((1, H, LB, DH), lambda b, i: (b, 0, i, 0)),
            pl.BlockSpec((1, H, LB, DH), lambda b, i: (b, 0, i, 0)),
            pl.BlockSpec((1, H, NPOOL, DH), lambda b, i: (b, 0, i, 0)),
            pl.BlockSpec((1, H, NPOOL, DH), lambda b, i: (b, 0, i, 0)),
        ],
        out_shape=[
            jax.ShapeDtypeStruct((B, H, L, DH), jnp.bfloat16),
            jax.ShapeDtypeStruct((B, H, L, DH), jnp.bfloat16),
            jax.ShapeDtypeStruct((B, H, L, DH), jnp.bfloat16),
            jax.ShapeDtypeStruct((B, H, NB, DH), jnp.float32),
            jax.ShapeDtypeStruct((B, H, NB, DH), jnp.float32),
        ],
        interpret=_INTERPRET,
    )(hidden, Wq, Wk, Wv)


def _scores_kernel(qp_ref, kp_ref, rq_ref, rk_ref, s_ref):
    qp = jnp.dot(qp_ref[0, 0], rq_ref[:, :], preferred_element_type=jnp.float32)
    kp = jnp.dot(kp_ref[0, 0], rk_ref[:, :], preferred_element_type=jnp.float32)
    s_ref[0, 0] = lax.dot_general(qp, kp, (((1,), (1,)), ((), ())),
                                  preferred_element_type=jnp.float32) * (1.0 / 8.0)


def _scores(q_pool, k_pool, Rq, Rk):
    grid = (B, H)
    r_spec = pl.BlockSpec((DH, DH), lambda b, h: (0, 0))
    return pl.pallas_call(
        _scores_kernel,
        grid=grid,
        in_specs=[
            pl.BlockSpec((1, 1, NB, DH), lambda b, h: (b, h, 0, 0)),
            pl.BlockSpec((1, 1, NB, DH), lambda b, h: (b, h, 0, 0)),
            r_spec, r_spec,
        ],
        out_specs=pl.BlockSpec((1, 1, NB, NB), lambda b, h: (b, h, 0, 0)),
        out_shape=jax.ShapeDtypeStruct((B, H, NB, NB), jnp.float32),
        interpret=_INTERPRET,
    )(q_pool, k_pool, Rq, Rk)


NROWS = B * H * NB
NWORK = 32
RW = NROWS // NWORK


def _topk_sc_kernel(s_hbm, sel_hbm, s_v, o_v, kbuf, vbuf):
    from jax.experimental.pallas import tpu_sc as plsc
    wid = lax.axis_index("s") * 2 + lax.axis_index("c")
    base = wid * RW
    pltpu.sync_copy(s_hbm.at[pl.ds(base, RW)], s_v)

    idx16 = lax.iota(jnp.int32, 16)

    def row(i, _):
        for pair, dst in ((0, 0), (1, 8)):
            for c, off in ((2 * pair, 16), (2 * pair + 1, 24)):
                ks, vs = plsc.sort_key_val(
                    s_v[i, pl.ds(c * 16, 16)], idx16 + 16 * c,
                    descending=True)
                kbuf[pl.ds(off, 16)] = ks
                vbuf[pl.ds(off, 16)] = vs
            km, vm = plsc.sort_key_val(
                kbuf[pl.ds(16, 16)], vbuf[pl.ds(16, 16)], descending=True)
            kbuf[pl.ds(dst, 16)] = km
            vbuf[pl.ds(dst, 16)] = vm
        _, vf = plsc.sort_key_val(
            kbuf[pl.ds(0, 16)], vbuf[pl.ds(0, 16)], descending=True)
        o_v[i, :] = vf
        return 0

    lax.fori_loop(0, RW, row, 0)
    pltpu.sync_copy(o_v, sel_hbm.at[pl.ds(base, RW)])


def _topk_sc(scores_flat):
    import functools as _ft
    from jax.experimental.pallas import tpu_sc as plsc
    mesh = plsc.VectorSubcoreMesh(core_axis_name="c", subcore_axis_name="s")
    kern = _ft.partial(
        pl.kernel,
        mesh=mesh,
        out_type=jax.ShapeDtypeStruct((NROWS, 16), jnp.int32),
        scratch_types=[
            pltpu.VMEM((RW, NB), jnp.float32),
            pltpu.VMEM((RW, 16), jnp.int32),
            pltpu.VMEM((40,), jnp.float32),
            pltpu.VMEM((40,), jnp.int32),
        ],
        compiler_params=pltpu.CompilerParams(needs_layout_passes=False),
    )(_topk_sc_kernel)
    return kern(scores_flat)


def _linstate_kernel(k_ref, v_ref, kv_ref, z_ref):
    pk = _phi(k_ref[0, 0].astype(jnp.float32))
    kv_ref[0, 0] = lax.dot_general(
        pk.astype(jnp.bfloat16), v_ref[0, 0], (((0,), (0,)), ((), ())),
        preferred_element_type=jnp.float32)
    z_ref[0, 0] = jnp.sum(pk, axis=0, keepdims=True)


def _linstate(k, v):
    grid = (B, H)
    return pl.pallas_call(
        _linstate_kernel,
        grid=grid,
        in_specs=[
            pl.BlockSpec((1, 1, L, DH), lambda b, h: (b, h, 0, 0)),
            pl.BlockSpec((1, 1, L, DH), lambda b, h: (b, h, 0, 0)),
        ],
        out_specs=[
            pl.BlockSpec((1, 1, DH, DH), lambda b, h: (b, h, 0, 0)),
            pl.BlockSpec((1, 1, 1, DH), lambda b, h: (b, h, 0, 0)),
        ],
        out_shape=[
            jax.ShapeDtypeStruct((B, H, DH, DH), jnp.float32),
            jax.ShapeDtypeStruct((B, H, 1, DH), jnp.float32),
        ],
        interpret=_INTERPRET,
    )(k, v)


GQ = 32


def _attn_kernel(sel_ref, alpha_ref, q_ref, k_ref, v_ref, kv_ref, z_ref,
                 o_ref):
    b = pl.program_id(0)
    h = pl.program_id(1)
    qi = pl.program_id(2)

    a = alpha_ref[h]
    kv = kv_ref[0, 0]
    z = z_ref[0, 0]
    for g in range(GQ):
        q_blk = q_ref[0, 0, g * BS:(g + 1) * BS, :]
        base = ((b * H + h) * NB + qi * GQ + g) * KK
        k_rows = []
        v_rows = []
        for j in range(KK):
            idx = sel_ref[base + j]
            k_rows.append(k_ref[0, 0, pl.ds(idx * BS, BS), :])
            v_rows.append(v_ref[0, 0, pl.ds(idx * BS, BS), :])
        k_cat = jnp.concatenate(k_rows, axis=0)
        v_cat = jnp.concatenate(v_rows, axis=0)
        s = lax.dot_general(q_blk, k_cat, (((1,), (1,)), ((), ())),
                            preferred_element_type=jnp.float32)
        m = jnp.max(s, axis=1, keepdims=True)
        e = jnp.exp(s - m)
        den_s = jnp.sum(e, axis=1, keepdims=True)
        o_s = jnp.dot(e.astype(jnp.bfloat16), v_cat,
                      preferred_element_type=jnp.float32) / den_s

        pq = _phi(q_blk.astype(jnp.float32))
        num = jnp.dot(pq, kv, preferred_element_type=jnp.float32)
        den_l = jnp.sum(pq * z, axis=1, keepdims=True) + 1e-6
        o_l = num / den_l

        o_ref[0, 0, g * BS:(g + 1) * BS, :] = (
            a * o_s + (1.0 - a) * o_l).astype(jnp.bfloat16)


def _attention(q, k, v, kv, z, sel_flat, alpha):
    grid = (B, H, NB // GQ)
    grid_spec = pltpu.PrefetchScalarGridSpec(
        num_scalar_prefetch=2,
        grid=grid,
        in_specs=[
            pl.BlockSpec((1, 1, GQ * BS, DH), lambda b, h, qi, *_: (b, h, qi, 0)),
            pl.BlockSpec((1, 1, L, DH), lambda b, h, qi, *_: (b, h, 0, 0)),
            pl.BlockSpec((1, 1, L, DH), lambda b, h, qi, *_: (b, h, 0, 0)),
            pl.BlockSpec((1, 1, DH, DH), lambda b, h, qi, *_: (b, h, 0, 0)),
            pl.BlockSpec((1, 1, 1, DH), lambda b, h, qi, *_: (b, h, 0, 0)),
        ],
        out_specs=pl.BlockSpec((1, 1, GQ * BS, DH),
                               lambda b, h, qi, *_: (b, h, qi, 0)),
    )
    return pl.pallas_call(
        _attn_kernel,
        grid_spec=grid_spec,
        out_shape=jax.ShapeDtypeStruct((B, H, L, DH), jnp.bfloat16),
        interpret=_INTERPRET,
    )(sel_flat, alpha, q, k, v, kv, z)


def _proj_kernel(x_ref, w_ref, o_ref):
    x = jnp.concatenate([x_ref[0, h] for h in range(H)], axis=1)
    o_ref[0] = jnp.dot(x, w_ref[:, :], preferred_element_type=jnp.float32)


def _out_proj(x, Wo):
    grid = (B, L // LB)
    return pl.pallas_call(
        _proj_kernel,
        grid=grid,
        in_specs=[
            pl.BlockSpec((1, H, LB, DH), lambda b, i: (b, 0, i, 0)),
            pl.BlockSpec((DM, DM), lambda b, i: (0, 0)),
        ],
        out_specs=pl.BlockSpec((1, LB, DM), lambda b, i: (b, i, 0)),
        out_shape=jax.ShapeDtypeStruct((B, L, DM), jnp.float32),
        interpret=_INTERPRET,
    )(x, Wo)


@jax.jit
def kernel(hidden_states, Wq, Wk, Wv, Wo, Rq, Rk, alpha_logits):
    q, k, v, q_pool, k_pool = _qkv(hidden_states, Wq, Wk, Wv)
    scores = _scores(q_pool, k_pool, Rq, Rk)
    sel_pad = _topk_sc(scores.reshape(NROWS, NB))
    sel_flat = sel_pad[:, :KK].reshape(-1)
    alpha = jax.nn.sigmoid(alpha_logits).reshape(H)
    kv, z = _linstate(k, v)
    out_attn = _attention(q, k, v, kv, z, sel_flat, alpha)
    return _out_proj(out_attn, Wo.astype(jnp.bfloat16))


# speedup vs baseline: 1.6631x; 1.0075x over previous
import functools

import jax
import jax.numpy as jnp
from jax import lax
from jax.experimental import pallas as pl
from jax.experimental.pallas import tpu as pltpu

B, L, DM, H, BS = 2, 4096, 1024, 16, 64
DH = DM // H
NB = L // BS
KK = 8
LB = 512
NPOOL = LB // BS

_INTERPRET = False


def _phi(x):
    return jnp.where(x > 0, x + 1.0, jnp.exp(x))


def _qkv_kernel(x_ref, wq_ref, wk_ref, wv_ref,
                q_ref, k_ref, v_ref, qp_ref, kp_ref):
    x = x_ref[0]
    q = jnp.dot(x, wq_ref[:, :], preferred_element_type=jnp.float32)
    k = jnp.dot(x, wk_ref[:, :], preferred_element_type=jnp.float32)
    v = jnp.dot(x, wv_ref[:, :], preferred_element_type=jnp.float32)
    scale = 1.0 / BS
    qp = jnp.stack([jnp.sum(q[j * BS:(j + 1) * BS, :], axis=0) * scale
                    for j in range(NPOOL)], axis=0)
    kp = jnp.stack([jnp.sum(k[j * BS:(j + 1) * BS, :], axis=0) * scale
                    for j in range(NPOOL)], axis=0)
    for h in range(H):
        sl = slice(h * DH, (h + 1) * DH)
        q_ref[0, h] = q[:, sl].astype(jnp.bfloat16)
        k_ref[0, h] = k[:, sl].astype(jnp.bfloat16)
        v_ref[0, h] = v[:, sl].astype(jnp.bfloat16)
        qp_ref[0, h] = qp[:, sl]
        kp_ref[0, h] = kp[:, sl]


def _qkv(hidden, Wq, Wk, Wv):
    grid = (B, L // LB)
    w_spec = pl.BlockSpec((DM, DM), lambda b, i: (0, 0))
    return pl.pallas_call(
        _qkv_kernel,
        grid=grid,
        in_specs=[
            pl.BlockSpec((1, LB, DM), lambda b, i: (b, i, 0)),
            w_spec, w_spec, w_spec,
        ],
        out_specs=[
            pl.BlockSpec((1, H, LB, DH), lambda b, i: (b, 0, i, 0)),
            pl.BlockSpec((1, H, LB, DH), lambda b, i: (b, 0, i, 0)),
            pl.BlockSpec((1, H, LB, DH), lambda b, i: (b, 0, i, 0)),
            pl.BlockSpec((1, H, NPOOL, DH), lambda b, i: (b, 0, i, 0)),
            pl.BlockSpec((1, H, NPOOL, DH), lambda b, i: (b, 0, i, 0)),
        ],
        out_shape=[
            jax.ShapeDtypeStruct((B, H, L, DH), jnp.bfloat16),
            jax.ShapeDtypeStruct((B, H, L, DH), jnp.bfloat16),
            jax.ShapeDtypeStruct((B, H, L, DH), jnp.bfloat16),
            jax.ShapeDtypeStruct((B, H, NB, DH), jnp.float32),
            jax.ShapeDtypeStruct((B, H, NB, DH), jnp.float32),
        ],
        interpret=_INTERPRET,
    )(hidden, Wq, Wk, Wv)


def _scores_kernel(qp_ref, kp_ref, rq_ref, rk_ref, s_ref):
    qp = jnp.dot(qp_ref[0, 0], rq_ref[:, :], preferred_element_type=jnp.float32)
    kp = jnp.dot(kp_ref[0, 0], rk_ref[:, :], preferred_element_type=jnp.float32)
    s_ref[0, 0] = lax.dot_general(qp, kp, (((1,), (1,)), ((), ())),
                                  preferred_element_type=jnp.float32) * (1.0 / 8.0)


def _scores(q_pool, k_pool, Rq, Rk):
    grid = (B, H)
    r_spec = pl.BlockSpec((DH, DH), lambda b, h: (0, 0))
    return pl.pallas_call(
        _scores_kernel,
        grid=grid,
        in_specs=[
            pl.BlockSpec((1, 1, NB, DH), lambda b, h: (b, h, 0, 0)),
            pl.BlockSpec((1, 1, NB, DH), lambda b, h: (b, h, 0, 0)),
            r_spec, r_spec,
        ],
        out_specs=pl.BlockSpec((1, 1, NB, NB), lambda b, h: (b, h, 0, 0)),
        out_shape=jax.ShapeDtypeStruct((B, H, NB, NB), jnp.float32),
        interpret=_INTERPRET,
    )(q_pool, k_pool, Rq, Rk)


NROWS = B * H * NB
NWORK = 32
RW = NROWS // NWORK


def _topk_sc_kernel(s_hbm, sel_hbm, s_v, o_v, kbuf, vbuf):
    from jax.experimental.pallas import tpu_sc as plsc
    wid = lax.axis_index("s") * 2 + lax.axis_index("c")
    base = wid * RW
    pltpu.sync_copy(s_hbm.at[pl.ds(base, RW)], s_v)

    idx16 = lax.iota(jnp.int32, 16)

    def row(i, _):
        for pair, dst in ((0, 0), (1, 8)):
            for c, off in ((2 * pair, 16), (2 * pair + 1, 24)):
                ks, vs = plsc.sort_key_val(
                    s_v[i, pl.ds(c * 16, 16)], idx16 + 16 * c,
                    descending=True)
                kbuf[pl.ds(off, 16)] = ks
                vbuf[pl.ds(off, 16)] = vs
            km, vm = plsc.sort_key_val(
                kbuf[pl.ds(16, 16)], vbuf[pl.ds(16, 16)], descending=True)
            kbuf[pl.ds(dst, 16)] = km
            vbuf[pl.ds(dst, 16)] = vm
        _, vf = plsc.sort_key_val(
            kbuf[pl.ds(0, 16)], vbuf[pl.ds(0, 16)], descending=True)
        o_v[i, :] = vf
        return 0

    lax.fori_loop(0, RW, row, 0)
    pltpu.sync_copy(o_v, sel_hbm.at[pl.ds(base, RW)])


def _topk_sc(scores_flat):
    import functools as _ft
    from jax.experimental.pallas import tpu_sc as plsc
    mesh = plsc.VectorSubcoreMesh(core_axis_name="c", subcore_axis_name="s")
    kern = _ft.partial(
        pl.kernel,
        mesh=mesh,
        out_type=jax.ShapeDtypeStruct((NROWS, 16), jnp.int32),
        scratch_types=[
            pltpu.VMEM((RW, NB), jnp.float32),
            pltpu.VMEM((RW, 16), jnp.int32),
            pltpu.VMEM((40,), jnp.float32),
            pltpu.VMEM((40,), jnp.int32),
        ],
        compiler_params=pltpu.CompilerParams(needs_layout_passes=False),
    )(_topk_sc_kernel)
    return kern(scores_flat)


def _linstate_kernel(k_ref, v_ref, kv_ref, z_ref):
    pk = _phi(k_ref[0, 0].astype(jnp.float32))
    kv_ref[0, 0] = lax.dot_general(
        pk.astype(jnp.bfloat16), v_ref[0, 0], (((0,), (0,)), ((), ())),
        preferred_element_type=jnp.float32)
    z_ref[0, 0] = jnp.sum(pk, axis=0, keepdims=True)


def _linstate(k, v):
    grid = (B, H)
    return pl.pallas_call(
        _linstate_kernel,
        grid=grid,
        in_specs=[
            pl.BlockSpec((1, 1, L, DH), lambda b, h: (b, h, 0, 0)),
            pl.BlockSpec((1, 1, L, DH), lambda b, h: (b, h, 0, 0)),
        ],
        out_specs=[
            pl.BlockSpec((1, 1, DH, DH), lambda b, h: (b, h, 0, 0)),
            pl.BlockSpec((1, 1, 1, DH), lambda b, h: (b, h, 0, 0)),
        ],
        out_shape=[
            jax.ShapeDtypeStruct((B, H, DH, DH), jnp.float32),
            jax.ShapeDtypeStruct((B, H, 1, DH), jnp.float32),
        ],
        interpret=_INTERPRET,
    )(k, v)


GQ = 64


def _attn_kernel(sel_ref, alpha_ref, q_ref, k_ref, v_ref, kv_ref, z_ref,
                 o_ref):
    b = pl.program_id(0)
    h = pl.program_id(1)
    qi = pl.program_id(2)

    a = alpha_ref[h]
    kv = kv_ref[0, 0]
    z = z_ref[0, 0]
    for g in range(GQ):
        q_blk = q_ref[0, 0, g * BS:(g + 1) * BS, :]
        base = ((b * H + h) * NB + qi * GQ + g) * KK
        k_rows = []
        v_rows = []
        for j in range(KK):
            idx = sel_ref[base + j]
            k_rows.append(k_ref[0, 0, pl.ds(idx * BS, BS), :])
            v_rows.append(v_ref[0, 0, pl.ds(idx * BS, BS), :])
        k_cat = jnp.concatenate(k_rows, axis=0)
        v_cat = jnp.concatenate(v_rows, axis=0)
        s = lax.dot_general(q_blk, k_cat, (((1,), (1,)), ((), ())),
                            preferred_element_type=jnp.float32)
        m = jnp.max(s, axis=1, keepdims=True)
        e = jnp.exp(s - m)
        den_s = jnp.sum(e, axis=1, keepdims=True)
        o_s = jnp.dot(e.astype(jnp.bfloat16), v_cat,
                      preferred_element_type=jnp.float32) / den_s

        pq = _phi(q_blk.astype(jnp.float32))
        num = jnp.dot(pq, kv, preferred_element_type=jnp.float32)
        den_l = jnp.sum(pq * z, axis=1, keepdims=True) + 1e-6
        o_l = num / den_l

        o_ref[0, 0, g * BS:(g + 1) * BS, :] = (
            a * o_s + (1.0 - a) * o_l).astype(jnp.bfloat16)


def _attention(q, k, v, kv, z, sel_flat, alpha):
    grid = (B, H, NB // GQ)
    grid_spec = pltpu.PrefetchScalarGridSpec(
        num_scalar_prefetch=2,
        grid=grid,
        in_specs=[
            pl.BlockSpec((1, 1, GQ * BS, DH), lambda b, h, qi, *_: (b, h, qi, 0)),
            pl.BlockSpec((1, 1, L, DH), lambda b, h, qi, *_: (b, h, 0, 0)),
            pl.BlockSpec((1, 1, L, DH), lambda b, h, qi, *_: (b, h, 0, 0)),
            pl.BlockSpec((1, 1, DH, DH), lambda b, h, qi, *_: (b, h, 0, 0)),
            pl.BlockSpec((1, 1, 1, DH), lambda b, h, qi, *_: (b, h, 0, 0)),
        ],
        out_specs=pl.BlockSpec((1, 1, GQ * BS, DH),
                               lambda b, h, qi, *_: (b, h, qi, 0)),
    )
    return pl.pallas_call(
        _attn_kernel,
        grid_spec=grid_spec,
        out_shape=jax.ShapeDtypeStruct((B, H, L, DH), jnp.bfloat16),
        interpret=_INTERPRET,
    )(sel_flat, alpha, q, k, v, kv, z)


def _proj_kernel(x_ref, w_ref, o_ref):
    x = jnp.concatenate([x_ref[0, h] for h in range(H)], axis=1)
    o_ref[0] = jnp.dot(x, w_ref[:, :], preferred_element_type=jnp.float32)


def _out_proj(x, Wo):
    grid = (B, L // LB)
    return pl.pallas_call(
        _proj_kernel,
        grid=grid,
        in_specs=[
            pl.BlockSpec((1, H, LB, DH), lambda b, i: (b, 0, i, 0)),
            pl.BlockSpec((DM, DM), lambda b, i: (0, 0)),
        ],
        out_specs=pl.BlockSpec((1, LB, DM), lambda b, i: (b, i, 0)),
        out_shape=jax.ShapeDtypeStruct((B, L, DM), jnp.float32),
        interpret=_INTERPRET,
    )(x, Wo)


@jax.jit
def kernel(hidden_states, Wq, Wk, Wv, Wo, Rq, Rk, alpha_logits):
    q, k, v, q_pool, k_pool = _qkv(hidden_states, Wq, Wk, Wv)
    scores = _scores(q_pool, k_pool, Rq, Rk)
    sel_pad = _topk_sc(scores.reshape(NROWS, NB))
    sel_flat = sel_pad[:, :KK].reshape(-1)
    alpha = jax.nn.sigmoid(alpha_logits).reshape(H)
    kv, z = _linstate(k, v)
    out_attn = _attention(q, k, v, kv, z, sel_flat, alpha)
    return _out_proj(out_attn, Wo.astype(jnp.bfloat16))
